# baseline (speedup 1.0000x reference)
import functools

import jax
import jax.numpy as jnp
from jax import lax
from jax.experimental import pallas as pl
from jax.experimental.pallas import tpu as pltpu
from jax.experimental.pallas import tpu_sc as plsc

_BB = 4
_HB = _BB // 2
_S = 8
_CH = 32


def _matvec_body(d0_ref, d1_ref, w_ref, b_ref, o_ref):
    for j, d_ref in enumerate((d0_ref, d1_ref)):
        for i in range(_HB):
            d = d_ref[i]
            w = w_ref[j * _HB + i]
            acc = jnp.sum(d * w, axis=1)
            o_ref[j * _HB + i] = acc[None, :] + b_ref[j * _HB + i, 0, 0]


def _tc_matvec(D, w, bias, nb):
    B, L, K = D.shape
    out = pl.pallas_call(
        _matvec_body,
        grid=(nb // _BB,),
        in_specs=[
            pl.BlockSpec((_HB, L, K), lambda b: (2 * b, 0, 0)),
            pl.BlockSpec((_HB, L, K), lambda b: (2 * b + 1, 0, 0)),
            pl.BlockSpec((_BB, 1, K), lambda b: (b, 0, 0)),
            pl.BlockSpec((_BB, 1, 128), lambda b: (b, 0, 0)),
        ],
        out_specs=pl.BlockSpec((_BB, 1, L), lambda b: (b, 0, 0)),
        out_shape=jax.ShapeDtypeStruct((nb, 1, L), jnp.float32),
    )(D, D, w, bias)
    return out.reshape(nb, L, 1)


def _sc_body(nbatch, rows_per_w, d_hbm, w_hbm, bias_hbm, out_hbm,
             dbuf, wbuf, bbuf, obuf, sem0, sem1):
    B, L, K = d_hbm.shape
    wid = lax.axis_index("s") * 2 + lax.axis_index("c")
    wpb = L // rows_per_w
    bsc = wid // wpb
    b = (B - nbatch) + bsc
    r0 = (wid % wpb) * rows_per_w

    pltpu.sync_copy(w_hbm.at[b], wbuf)
    pltpu.sync_copy(bias_hbm.at[b], bbuf)

    def _xlane_sum(v):
        dn = lax.GatherDimensionNumbers(
            offset_dims=(), collapsed_slice_dims=(0,), start_index_map=(0,))
        for sh in (8, 4, 2, 1):
            idx = lax.iota(jnp.int32, 16) ^ sh
            v = v + lax.gather(v, idx[:, None], dn, slice_sizes=(1,),
                               mode=lax.GatherScatterMode.PROMISE_IN_BOUNDS)
        return v

    nch = rows_per_w // _CH
    sems = (sem0, sem1)
    cps = {}
    cps[0] = pltpu.async_copy(d_hbm.at[b, pl.ds(r0, _CH)], dbuf.at[0], sem0)
    for c in range(nch):
        cur = c % 2
        if c + 1 < nch:
            nxt = (c + 1) % 2
            cps[nxt] = pltpu.async_copy(
                d_hbm.at[b, pl.ds(r0 + (c + 1) * _CH, _CH)],
                dbuf.at[nxt], sems[nxt])
        cps[cur].wait()
        lanes = lax.iota(jnp.int32, 16)

        def row_group(g, _):
            def one_row(r, res):
                row = g * 16 + r

                def kc_body(kc, acc):
                    for j in range(8):
                        off = (kc * 8 + j) * 16
                        acc = acc + (dbuf[cur, row, pl.ds(off, 16)]
                                     * wbuf[pl.ds(off, 16)])
                    return acc

                acc = lax.fori_loop(0, K // 128, kc_body,
                                    jnp.zeros((16,), jnp.float32))
                s = _xlane_sum(acc)
                return jnp.where(lanes == r, s, res)

            res = lax.fori_loop(0, 16, one_row, jnp.zeros((16,), jnp.float32))
            obuf[pl.ds(c * _CH + g * 16, 16)] = res + bbuf[...]
            return _

        lax.fori_loop(0, _CH // 16, row_group, jnp.int32(0))

    pltpu.sync_copy(obuf, out_hbm.at[bsc, pl.ds(r0, rows_per_w)])


def _sc_matvec(D, coef, nbatch):
    B, L, K = D.shape
    w = coef[:, :K]
    bias = jnp.broadcast_to(coef[:, K:], (B, 16))
    rows_per_w = (nbatch * L) // 32
    mesh = plsc.VectorSubcoreMesh(core_axis_name="c", subcore_axis_name="s")
    kern = functools.partial(
        pl.kernel,
        out_type=jax.ShapeDtypeStruct((nbatch, L), jnp.float32),
        mesh=mesh,
        scratch_types=[
            pltpu.VMEM((2, _CH, K), jnp.float32),
            pltpu.VMEM((K,), jnp.float32),
            pltpu.VMEM((16,), jnp.float32),
            pltpu.VMEM((rows_per_w,), jnp.float32),
            pltpu.SemaphoreType.DMA,
            pltpu.SemaphoreType.DMA,
        ],
    )(functools.partial(_sc_body, nbatch, rows_per_w))
    return kern(D, w, bias)


def kernel(dict, coef):
    D = dict
    B, L, K = D.shape
    w = coef[:, :K].reshape(B, 1, K)
    bias = jnp.broadcast_to(coef[:, K:].reshape(B, 1, 1), (B, 1, 128))

    nb_tc = B - _S
    out_sc = _sc_matvec(D, coef, _S)
    out_tc = _tc_matvec(D, w, bias, nb_tc)
    return jnp.concatenate([out_tc, out_sc.reshape(_S, L, 1)], axis=0)


# speedup vs baseline: 1.0258x; 1.0258x over previous
import functools

import jax
import jax.numpy as jnp
from jax import lax
from jax.experimental import pallas as pl
from jax.experimental.pallas import tpu as pltpu
from jax.experimental.pallas import tpu_sc as plsc

_BB = 4
_HB = _BB // 2
_S = 8
_CH = 32
_KP = 1040


def _matvec_body(d0_ref, d1_ref, c_ref, o_ref):
    for j, d_ref in enumerate((d0_ref, d1_ref)):
        for i in range(_HB):
            bi = j * _HB + i
            d = d_ref[i]
            w = c_ref[bi, :, 0:1024]
            acc = jnp.sum(d * w, axis=1)
            o_ref[bi] = acc[None, :] + c_ref[bi, 0, 1024]


def _tc_matvec(D, coef_pad, nb):
    B, L, K = D.shape
    c3 = coef_pad.reshape(B, 1, _KP)
    out = pl.pallas_call(
        _matvec_body,
        grid=(nb // _BB,),
        in_specs=[
            pl.BlockSpec((_HB, L, K), lambda b: (2 * b, 0, 0)),
            pl.BlockSpec((_HB, L, K), lambda b: (2 * b + 1, 0, 0)),
            pl.BlockSpec((_BB, 1, _KP), lambda b: (b, 0, 0)),
        ],
        out_specs=pl.BlockSpec((_BB, 1, L), lambda b: (b, 0, 0)),
        out_shape=jax.ShapeDtypeStruct((nb, 1, L), jnp.float32),
    )(D, D, c3)
    return out.reshape(nb, L, 1)


def _lane_perm(v, idx):
    dn = lax.GatherDimensionNumbers(
        offset_dims=(), collapsed_slice_dims=(0,), start_index_map=(0,))
    return lax.gather(v, idx[:, None], dn, slice_sizes=(1,),
                      mode=lax.GatherScatterMode.PROMISE_IN_BOUNDS)


def _sc_body(nbatch, rows_per_w, d_hbm, c_hbm, out_hbm,
             dbuf, wbuf, bbuf, obuf, sem0, sem1):
    B, L, K = d_hbm.shape
    wid = lax.axis_index("s") * 2 + lax.axis_index("c")
    wpb = L // rows_per_w
    bsc = wid // wpb
    b = (B - nbatch) + bsc
    r0 = (wid % wpb) * rows_per_w

    pltpu.sync_copy(c_hbm.at[b, pl.ds(0, K)], wbuf)
    pltpu.sync_copy(c_hbm.at[b, pl.ds(K, 16)], bbuf)
    lanes = lax.iota(jnp.int32, 16)
    bias = _lane_perm(bbuf[...], jnp.zeros((16,), jnp.int32))

    def _xlane_sum(v):
        for sh in (8, 4, 2, 1):
            v = v + _lane_perm(v, lanes ^ sh)
        return v

    nch = rows_per_w // _CH
    sems = (sem0, sem1)
    cps = {}
    cps[0] = pltpu.async_copy(d_hbm.at[b, pl.ds(r0, _CH)], dbuf.at[0], sem0)
    for c in range(nch):
        cur = c % 2
        if c + 1 < nch:
            nxt = (c + 1) % 2
            cps[nxt] = pltpu.async_copy(
                d_hbm.at[b, pl.ds(r0 + (c + 1) * _CH, _CH)],
                dbuf.at[nxt], sems[nxt])
        cps[cur].wait()

        def row_group(g, _):
            def quad(q, res):
                row = g * 16 + q * 4

                def kc_body(kc, accs):
                    a0, a1, a2, a3 = accs
                    for j in range(8):
                        off = (kc * 8 + j) * 16
                        wv = wbuf[pl.ds(off, 16)]
                        a0 = a0 + dbuf[cur, row, pl.ds(off, 16)] * wv
                        a1 = a1 + dbuf[cur, row + 1, pl.ds(off, 16)] * wv
                        a2 = a2 + dbuf[cur, row + 2, pl.ds(off, 16)] * wv
                        a3 = a3 + dbuf[cur, row + 3, pl.ds(off, 16)] * wv
                    return a0, a1, a2, a3

                z = jnp.zeros((16,), jnp.float32)
                accs = lax.fori_loop(0, K // 128, kc_body, (z, z, z, z))
                for t in range(4):
                    s = _xlane_sum(accs[t])
                    res = jnp.where(lanes == q * 4 + t, s, res)
                return res

            res = lax.fori_loop(0, 4, quad, jnp.zeros((16,), jnp.float32))
            obuf[pl.ds(c * _CH + g * 16, 16)] = res + bias
            return _

        lax.fori_loop(0, _CH // 16, row_group, jnp.int32(0))

    pltpu.sync_copy(obuf, out_hbm.at[bsc, pl.ds(r0, rows_per_w)])


def _sc_matvec(D, coef_pad, nbatch):
    B, L, K = D.shape
    rows_per_w = (nbatch * L) // 32
    mesh = plsc.VectorSubcoreMesh(core_axis_name="c", subcore_axis_name="s")
    kern = functools.partial(
        pl.kernel,
        out_type=jax.ShapeDtypeStruct((nbatch, L), jnp.float32),
        mesh=mesh,
        scratch_types=[
            pltpu.VMEM((2, _CH, K), jnp.float32),
            pltpu.VMEM((K,), jnp.float32),
            pltpu.VMEM((16,), jnp.float32),
            pltpu.VMEM((rows_per_w,), jnp.float32),
            pltpu.SemaphoreType.DMA,
            pltpu.SemaphoreType.DMA,
        ],
    )(functools.partial(_sc_body, nbatch, rows_per_w))
    return kern(D, coef_pad)


def kernel(dict, coef):
    D = dict
    B, L, K = D.shape
    coef_pad = jnp.pad(coef, ((0, 0), (0, _KP - coef.shape[1])))

    nb_tc = B - _S
    out_sc = _sc_matvec(D, coef_pad, _S)
    out_tc = _tc_matvec(D, coef_pad, nb_tc)
    return jnp.concatenate([out_tc, out_sc.reshape(_S, L, 1)], axis=0)
